# Initial kernel scaffold; baseline (speedup 1.0000x reference)
import functools

import jax
import jax.numpy as jnp
from jax import lax
from jax.experimental import pallas as pl
from jax.experimental.pallas import tpu as pltpu
from jax.experimental.pallas import tpu_sc as plsc

NEG = -1e30
N_PAD = 10240
C1 = 384
C2 = 576
BN_A = 256
BN_B = 512
BN_C = 512


def _tc_a_body(x_ref, w_ref, asv_ref, adv_ref, h_ref, as_ref, ad_ref):
    h = jnp.dot(x_ref[...], w_ref[...], preferred_element_type=jnp.float32)
    h_ref[0] = h
    h4 = h.reshape(BN_A, 4, 32)
    as_ref[...] = (h4 * asv_ref[...][None]).sum(-1)
    ad_ref[...] = (h4 * adv_ref[...][None]).sum(-1)


def _tc_a(x_p, w_p, asv, adv):
    grid = (2, N_PAD // BN_A)
    return pl.pallas_call(
        _tc_a_body,
        grid=grid,
        in_specs=[
            pl.BlockSpec((BN_A, 768), lambda c, i: (i, 0)),
            pl.BlockSpec((768, 128), lambda c, i: (0, c)),
            pl.BlockSpec((4, 32), lambda c, i: (c, 0)),
            pl.BlockSpec((4, 32), lambda c, i: (c, 0)),
        ],
        out_specs=[
            pl.BlockSpec((1, BN_A, 128), lambda c, i: (c, i, 0)),
            pl.BlockSpec((BN_A, 4), lambda c, i: (i, c)),
            pl.BlockSpec((BN_A, 4), lambda c, i: (i, c)),
        ],
        out_shape=[
            jax.ShapeDtypeStruct((2, N_PAD, 128), jnp.float32),
            jax.ShapeDtypeStruct((N_PAD, 8), jnp.float32),
            jax.ShapeDtypeStruct((N_PAD, 8), jnp.float32),
        ],
    )(x_p, w_p, asv, adv)


def _sc1_body(src_ref, dst_ref, ta_ref, tb_ref, hcat_ref,
              u_out, den_out,
              u_sp, den_sp, sidx, didx, gidx, arows, brows, exr, rrows,
              zbuf, sem):
    core = lax.axis_index("c")
    sub = lax.axis_index("s")
    T = src_ref.shape[0] // 16
    rows_per_tile = N_PAD // 16
    rs = sub * rows_per_tile

    for r in range(8):
        for v in range(8):
            zbuf[r, pl.ds(16 * v, 16)] = jnp.zeros((16,), jnp.float32)

    def zloop(k, _):
        pltpu.sync_copy(zbuf, u_sp.at[pl.ds(rs + k * 8, 8)])
        return 0
    lax.fori_loop(0, rows_per_tile // 8, zloop, 0)

    def zloop2(k, _):
        pltpu.sync_copy(zbuf.at[:, 0:16], den_sp.at[pl.ds(rs + k * 8, 8)])
        return 0
    lax.fori_loop(0, rows_per_tile // 8, zloop2, 0)
    plsc.subcore_barrier()

    noff = core * N_PAD

    def chunk(k, _):
        base = sub * T + k * C1
        pltpu.sync_copy(src_ref.at[pl.ds(base, C1)], sidx)
        pltpu.sync_copy(dst_ref.at[pl.ds(base, C1)], didx)

        def addl(i, _):
            gidx[pl.ds(i * 16, 16)] = sidx[pl.ds(i * 16, 16)] + noff
            return 0
        lax.fori_loop(0, C1 // 16, addl, 0)

        c1 = pltpu.async_copy(ta_ref.at[sidx], arows, sem)
        c2 = pltpu.async_copy(tb_ref.at[didx], brows, sem)
        c3 = pltpu.async_copy(hcat_ref.at[gidx], rrows, sem)
        c1.wait()
        c2.wait()
        c3.wait()

        def edge(i, _):
            s = arows[i, :] + brows[i, :]
            ex = jnp.exp(jnp.maximum(s, 0.2 * s))
            exr[i, :] = ex
            for j in range(4):
                w = plsc.load_gather(
                    exr,
                    [jnp.full((16,), i, jnp.int32),
                     jnp.full((16,), 4 * core + j, jnp.int32)])
                for t in range(2):
                    sl = pl.ds((2 * j + t) * 16, 16)
                    rrows[i, sl] = rrows[i, sl] * w
            return 0
        lax.fori_loop(0, C1, edge, 0)

        @pl.when(core == 0)
        def _():
            pltpu.sync_copy(exr, den_sp.at[didx], add=True)

        pltpu.sync_copy(rrows, u_sp.at[didx], add=True)
        return 0

    lax.fori_loop(0, T // C1, chunk, 0)
    plsc.subcore_barrier()

    pltpu.sync_copy(u_sp.at[pl.ds(rs, rows_per_tile)],
                    u_out.at[core, pl.ds(rs, rows_per_tile)])

    @pl.when(core == 0)
    def _():
        pltpu.sync_copy(den_sp.at[pl.ds(rs, rows_per_tile)],
                        den_out.at[pl.ds(rs, rows_per_tile)])


def _sc1(src_p, dst_p, ta, tb, hcat):
    mesh = plsc.VectorSubcoreMesh(core_axis_name="c", subcore_axis_name="s")
    f = pl.kernel(
        _sc1_body,
        out_type=[
            jax.ShapeDtypeStruct((2, N_PAD, 128), jnp.float32),
            jax.ShapeDtypeStruct((N_PAD, 16), jnp.float32),
        ],
        mesh=mesh,
        scratch_types=[
            pltpu.VMEM_SHARED((N_PAD, 128), jnp.float32),
            pltpu.VMEM_SHARED((N_PAD, 16), jnp.float32),
            pltpu.VMEM((C1,), jnp.int32),
            pltpu.VMEM((C1,), jnp.int32),
            pltpu.VMEM((C1,), jnp.int32),
            pltpu.VMEM((C1, 16), jnp.float32),
            pltpu.VMEM((C1, 16), jnp.float32),
            pltpu.VMEM((C1, 16), jnp.float32),
            pltpu.VMEM((C1, 128), jnp.float32),
            pltpu.VMEM((8, 128), jnp.float32),
            pltpu.SemaphoreType.DMA,
        ],
    )
    return f(src_p, dst_p, ta, tb, hcat)


def _tc_b_body(u_ref, d_ref, b1_ref, w2_ref, as2_ref, ad2_ref,
               h2_ref, ta2_ref, tb2_ref):
    i = pl.program_id(0)
    u = u_ref[...]
    h1 = jnp.concatenate([u[0], u[1]], axis=-1)
    den = d_ref[...][:, :8]
    drep = jnp.broadcast_to(den[:, :, None], (BN_B, 8, 32)).reshape(BN_B, 256)
    h1 = h1 / (drep + 1e-16) + b1_ref[...]
    h1 = jnp.where(h1 > 0, h1, jnp.exp(h1) - 1.0)
    h2 = jnp.dot(h1, w2_ref[...], preferred_element_type=jnp.float32)
    h2_ref[...] = h2
    asrc2 = (h2 * as2_ref[...]).sum(-1)
    adst2 = (h2 * ad2_ref[...]).sum(-1)
    col = lax.broadcasted_iota(jnp.int32, (BN_B, 16), 1)
    rown = lax.broadcasted_iota(jnp.int32, (BN_B, 16), 0) + i * BN_B
    ta2_ref[...] = jnp.where(col == 0, asrc2[:, None], NEG)
    tb2_ref[...] = jnp.where(col == 0,
                             jnp.where(rown < 10000, adst2[:, None], NEG),
                             0.0)


def _tc_b(u1, den1, b1r, w2p, as2v, ad2v):
    grid = (N_PAD // BN_B,)
    return pl.pallas_call(
        _tc_b_body,
        grid=grid,
        in_specs=[
            pl.BlockSpec((2, BN_B, 128), lambda i: (0, i, 0)),
            pl.BlockSpec((BN_B, 16), lambda i: (i, 0)),
            pl.BlockSpec((1, 256), lambda i: (0, 0)),
            pl.BlockSpec((256, 16), lambda i: (0, 0)),
            pl.BlockSpec((1, 16), lambda i: (0, 0)),
            pl.BlockSpec((1, 16), lambda i: (0, 0)),
        ],
        out_specs=[
            pl.BlockSpec((BN_B, 16), lambda i: (i, 0)),
            pl.BlockSpec((BN_B, 16), lambda i: (i, 0)),
            pl.BlockSpec((BN_B, 16), lambda i: (i, 0)),
        ],
        out_shape=[
            jax.ShapeDtypeStruct((N_PAD, 16), jnp.float32),
            jax.ShapeDtypeStruct((N_PAD, 16), jnp.float32),
            jax.ShapeDtypeStruct((N_PAD, 16), jnp.float32),
        ],
    )(u1, den1, b1r, w2p, as2v, ad2v)


def _sc2_body(src_ref, dst_ref, ta2_ref, tb2_ref, h2_ref,
              u2_out, d2_out,
              u2_sp, d2_sp, sidx, didx, arows, brows, hrows, exr, zbuf, sem):
    core = lax.axis_index("c")
    sub = lax.axis_index("s")
    half = src_ref.shape[0] // 2
    T = half // 16
    rows_per_tile = N_PAD // 16
    rs = sub * rows_per_tile

    for r in range(8):
        zbuf[r, :] = jnp.zeros((16,), jnp.float32)

    def zloop(k, _):
        pltpu.sync_copy(zbuf, u2_sp.at[pl.ds(rs + k * 8, 8)])
        pltpu.sync_copy(zbuf, d2_sp.at[pl.ds(rs + k * 8, 8)])
        return 0
    lax.fori_loop(0, rows_per_tile // 8, zloop, 0)
    plsc.subcore_barrier()

    def chunk(k, _):
        base = core * half + sub * T + k * C2
        pltpu.sync_copy(src_ref.at[pl.ds(base, C2)], sidx)
        pltpu.sync_copy(dst_ref.at[pl.ds(base, C2)], didx)
        c1 = pltpu.async_copy(ta2_ref.at[sidx], arows, sem)
        c2 = pltpu.async_copy(tb2_ref.at[didx], brows, sem)
        c3 = pltpu.async_copy(h2_ref.at[sidx], hrows, sem)
        c1.wait()
        c2.wait()
        c3.wait()

        def edge(i, _):
            s = arows[i, :] + brows[i, :]
            ex = jnp.exp(jnp.maximum(s, 0.2 * s))
            exr[i, :] = ex
            w = plsc.load_gather(
                exr,
                [jnp.full((16,), i, jnp.int32),
                 jnp.full((16,), 0, jnp.int32)])
            hrows[i, :] = hrows[i, :] * w
            return 0
        lax.fori_loop(0, C2, edge, 0)

        pltpu.sync_copy(exr, d2_sp.at[didx], add=True)
        pltpu.sync_copy(hrows, u2_sp.at[didx], add=True)
        return 0

    lax.fori_loop(0, T // C2, chunk, 0)
    plsc.subcore_barrier()

    pltpu.sync_copy(u2_sp.at[pl.ds(rs, rows_per_tile)],
                    u2_out.at[core, pl.ds(rs, rows_per_tile)])
    pltpu.sync_copy(d2_sp.at[pl.ds(rs, rows_per_tile)],
                    d2_out.at[core, pl.ds(rs, rows_per_tile)])


def _sc2(src_p, dst_p, ta2, tb2, h2):
    mesh = plsc.VectorSubcoreMesh(core_axis_name="c", subcore_axis_name="s")
    f = pl.kernel(
        _sc2_body,
        out_type=[
            jax.ShapeDtypeStruct((2, N_PAD, 16), jnp.float32),
            jax.ShapeDtypeStruct((2, N_PAD, 16), jnp.float32),
        ],
        mesh=mesh,
        scratch_types=[
            pltpu.VMEM_SHARED((N_PAD, 16), jnp.float32),
            pltpu.VMEM_SHARED((N_PAD, 16), jnp.float32),
            pltpu.VMEM((C2,), jnp.int32),
            pltpu.VMEM((C2,), jnp.int32),
            pltpu.VMEM((C2, 16), jnp.float32),
            pltpu.VMEM((C2, 16), jnp.float32),
            pltpu.VMEM((C2, 16), jnp.float32),
            pltpu.VMEM((C2, 16), jnp.float32),
            pltpu.VMEM((8, 16), jnp.float32),
            pltpu.SemaphoreType.DMA,
        ],
    )
    return f(src_p, dst_p, ta2, tb2, h2)


def _tc_c_body(u_ref, d_ref, b2_ref, o_ref):
    u = u_ref[...]
    d = d_ref[...]
    usum = u[0] + u[1]
    dsum = d[0][:, 0:1] + d[1][:, 0:1]
    o_ref[...] = usum / (dsum + 1e-16) + b2_ref[...]


def _tc_c(u2p, d2p, b2r):
    grid = (N_PAD // BN_C,)
    return pl.pallas_call(
        _tc_c_body,
        grid=grid,
        in_specs=[
            pl.BlockSpec((2, BN_C, 16), lambda i: (0, i, 0)),
            pl.BlockSpec((2, BN_C, 16), lambda i: (0, i, 0)),
            pl.BlockSpec((1, 16), lambda i: (0, 0)),
        ],
        out_specs=pl.BlockSpec((BN_C, 16), lambda i: (i, 0)),
        out_shape=jax.ShapeDtypeStruct((N_PAD, 16), jnp.float32),
    )(u2p, d2p, b2r)


def kernel(x, edge_index, W1, att_src1, att_dst1, b1, W2, att_src2,
           att_dst2, b2):
    N = x.shape[0]
    E0 = edge_index.shape[1]
    loops = jnp.arange(N, dtype=edge_index.dtype)
    ei = jnp.concatenate([edge_index, jnp.stack([loops, loops])], axis=1)
    src, dst = ei[0], ei[1]
    E = E0 + N
    step = 16 * C1
    e_pad = ((E + step - 1) // step) * step
    assert e_pad % (32 * C2) == 0

    src_p = jnp.concatenate([src, jnp.zeros((e_pad - E,), jnp.int32)])
    dst_p = jnp.concatenate([dst, jnp.full((e_pad - E,), N, jnp.int32)])

    x_p = jnp.zeros((N_PAD, 768), jnp.float32).at[:N, :767].set(x)
    w1p = jnp.zeros((768, 256), jnp.float32).at[:767].set(W1)
    h_split, a_src, a_dst = _tc_a(x_p, w1p, att_src1.reshape(8, 32),
                                  att_dst1.reshape(8, 32))
    hcat = h_split.reshape(2 * N_PAD, 128)

    ta = jnp.full((N_PAD, 16), NEG, jnp.float32).at[:, :8].set(a_src)
    tb = (jnp.zeros((N_PAD, 16), jnp.float32).at[:, :8].set(a_dst)
          .at[N:, :].set(NEG))

    u1, den1 = _sc1(src_p, dst_p, ta, tb, hcat)

    w2p = jnp.zeros((256, 16), jnp.float32).at[:, :10].set(W2)
    as2v = jnp.zeros((1, 16), jnp.float32).at[0, :10].set(att_src2.reshape(10))
    ad2v = jnp.zeros((1, 16), jnp.float32).at[0, :10].set(att_dst2.reshape(10))
    h2, ta2, tb2 = _tc_b(u1, den1, b1.reshape(1, 256), w2p, as2v, ad2v)

    u2p, d2p = _sc2(src_p, dst_p, ta2, tb2, h2)

    b2r = jnp.zeros((1, 16), jnp.float32).at[0, :10].set(b2)
    outp = _tc_c(u2p, d2p, b2r)
    return outp[:N, :10]


# speedup vs baseline: 33.5742x; 33.5742x over previous
import functools

import jax
import jax.numpy as jnp
from jax import lax
from jax.experimental import pallas as pl
from jax.experimental.pallas import tpu as pltpu
from jax.experimental.pallas import tpu_sc as plsc

NEG = -1e30
N_PAD = 10240
C1 = 192
C2 = 288
ZR = 64
BN_A = 256
BN_B = 512
BN_C = 512


def _tc_a_body(x_ref, w_ref, asv_ref, adv_ref, h_ref, as_ref, ad_ref):
    h = jnp.dot(x_ref[...], w_ref[...], preferred_element_type=jnp.float32)
    h_ref[0] = h
    h4 = h.reshape(BN_A, 4, 32)
    as_ref[0] = (h4 * asv_ref[...]).sum(-1)
    ad_ref[0] = (h4 * adv_ref[...]).sum(-1)


def _tc_a(x_p, w_p, asv, adv):
    grid = (2, N_PAD // BN_A)
    return pl.pallas_call(
        _tc_a_body,
        grid=grid,
        in_specs=[
            pl.BlockSpec((BN_A, 768), lambda c, i: (i, 0)),
            pl.BlockSpec((768, 128), lambda c, i: (0, c)),
            pl.BlockSpec((1, 4, 32), lambda c, i: (c, 0, 0)),
            pl.BlockSpec((1, 4, 32), lambda c, i: (c, 0, 0)),
        ],
        out_specs=[
            pl.BlockSpec((1, BN_A, 128), lambda c, i: (c, i, 0)),
            pl.BlockSpec((1, BN_A, 4), lambda c, i: (c, i, 0)),
            pl.BlockSpec((1, BN_A, 4), lambda c, i: (c, i, 0)),
        ],
        out_shape=[
            jax.ShapeDtypeStruct((2, N_PAD, 128), jnp.float32),
            jax.ShapeDtypeStruct((2, N_PAD, 4), jnp.float32),
            jax.ShapeDtypeStruct((2, N_PAD, 4), jnp.float32),
        ],
    )(x_p, w_p, asv, adv)


def _sc1_body(src_ref, dst_ref, comb0_ref, comb1_ref, hcat_ref,
              u_out, den_out,
              u_sp, den_sp, comb_sp, t512, zidx8, bflat, sidx, didx, gidx,
              si4, di4, exf, rrows, zidx, sem, sem2):
    core = lax.axis_index("c")
    sub = lax.axis_index("s")
    T = src_ref.shape[0] // 16
    rpt = N_PAD // 16
    rs = sub * rpt
    iota16 = lax.broadcasted_iota(jnp.int32, (16,), 0)
    e16 = iota16 // 4
    j16 = iota16 % 4
    zero16 = jnp.zeros((16,), jnp.float32)
    zi16 = jnp.zeros((16,), jnp.int32)

    def fillc(c, _):
        off = rs * 8 + c * 512
        @pl.when(core == 0)
        def _():
            pltpu.sync_copy(comb0_ref.at[pl.ds(off, 512)], t512)

        @pl.when(core == 1)
        def _():
            pltpu.sync_copy(comb1_ref.at[pl.ds(off, 512)], t512)

        for v in range(32):
            zidx8[pl.ds(16 * v, 16)] = iota16 + (off + 16 * v)
        pltpu.sync_copy(t512, comb_sp.at[zidx8])
        return 0
    lax.fori_loop(0, rpt * 8 // 512, fillc, 0)

    def zt(i, _):
        t512[pl.ds(16 * i, 16)] = zero16
        return 0
    lax.fori_loop(0, 32, zt, 0)

    def zden(c, _):
        off = rs * 8 + c * 512
        for v in range(32):
            zidx8[pl.ds(16 * v, 16)] = iota16 + (off + 16 * v)
        pltpu.sync_copy(t512, den_sp.at[zidx8])
        return 0
    lax.fori_loop(0, rpt * 8 // 512, zden, 0)

    def zsrc(r, _):
        for v in range(8):
            rrows[r, pl.ds(16 * v, 16)] = zero16
        return 0
    lax.fori_loop(0, ZR, zsrc, 0)

    def zscat(k, _):
        b = rs + k * ZR
        for v in range(ZR // 16):
            zidx[pl.ds(16 * v, 16)] = iota16 + (b + 16 * v)
        pltpu.sync_copy(rrows.at[pl.ds(0, ZR)], u_sp.at[zidx])
        return 0
    lax.fori_loop(0, rpt // ZR, zscat, 0)
    plsc.subcore_barrier()

    noff = core * N_PAD

    def chunk(k, _):
        base = sub * T + k * C1
        pltpu.sync_copy(src_ref.at[pl.ds(base, C1)], sidx)
        pltpu.sync_copy(dst_ref.at[pl.ds(base, C1)], didx)

        def addl(i, _):
            gidx[pl.ds(i * 16, 16)] = sidx[pl.ds(i * 16, 16)] + noff
            return 0
        lax.fori_loop(0, C1 // 16, addl, 0)

        def mkidx(i, _):
            si = plsc.load_gather(sidx, [4 * i + e16])
            di = plsc.load_gather(didx, [4 * i + e16])
            si4[pl.ds(16 * i, 16)] = si * 8 + j16
            di4[pl.ds(16 * i, 16)] = di * 8 + (j16 + 4)
            return 0
        lax.fori_loop(0, C1 // 4, mkidx, 0)

        ga = pltpu.async_copy(comb_sp.at[si4], exf, sem)
        gb = pltpu.async_copy(comb_sp.at[di4], bflat, sem)
        gh = pltpu.async_copy(hcat_ref.at[gidx], rrows, sem2)
        ga.wait()
        gb.wait()

        def quad(i, _):
            a = exf[pl.ds(16 * i, 16)]
            b = bflat[pl.ds(16 * i, 16)]
            s = a + b
            ex = jnp.exp(jnp.maximum(s, 0.2 * s))
            exf[pl.ds(16 * i, 16)] = ex
            return 0
        lax.fori_loop(0, C1 // 4, quad, 0)
        gh.wait()

        def edge(r, _):
            for jj in range(4):
                w = plsc.load_gather(exf, [zi16 + (r * 4 + jj)])
                sl0 = pl.ds(32 * jj, 16)
                sl1 = pl.ds(32 * jj + 16, 16)
                rrows[r, sl0] = rrows[r, sl0] * w
                rrows[r, sl1] = rrows[r, sl1] * w
            return 0
        lax.fori_loop(0, C1, edge, 0)

        pltpu.sync_copy(exf, den_sp.at[di4], add=True)
        pltpu.sync_copy(rrows, u_sp.at[didx], add=True)
        return 0

    lax.fori_loop(0, T // C1, chunk, 0)
    plsc.subcore_barrier()

    pltpu.sync_copy(u_sp.at[pl.ds(rs, rpt)],
                    u_out.at[core, pl.ds(rs, rpt)])
    pltpu.sync_copy(den_sp.at[pl.ds(rs * 8, rpt * 8)],
                    den_out.at[core, pl.ds(rs * 8, rpt * 8)])


def _sc1(src_p, dst_p, comb0, comb1, hcat):
    mesh = plsc.VectorSubcoreMesh(core_axis_name="c", subcore_axis_name="s")
    f = pl.kernel(
        _sc1_body,
        out_type=[
            jax.ShapeDtypeStruct((2, N_PAD, 128), jnp.float32),
            jax.ShapeDtypeStruct((2, N_PAD * 8), jnp.float32),
        ],
        mesh=mesh,
        scratch_types=[
            pltpu.VMEM_SHARED((N_PAD, 128), jnp.float32),
            pltpu.VMEM_SHARED((N_PAD * 8,), jnp.float32),
            pltpu.VMEM_SHARED((N_PAD * 8,), jnp.float32),
            pltpu.VMEM((512,), jnp.float32),
            pltpu.VMEM((512,), jnp.int32),
            pltpu.VMEM((C1 * 4,), jnp.float32),
            pltpu.VMEM((C1,), jnp.int32),
            pltpu.VMEM((C1,), jnp.int32),
            pltpu.VMEM((C1,), jnp.int32),
            pltpu.VMEM((C1 * 4,), jnp.int32),
            pltpu.VMEM((C1 * 4,), jnp.int32),
            pltpu.VMEM((C1 * 4,), jnp.float32),
            pltpu.VMEM((C1, 128), jnp.float32),
            pltpu.VMEM((ZR,), jnp.int32),
            pltpu.SemaphoreType.DMA,
            pltpu.SemaphoreType.DMA,
        ],
        compiler_params=pltpu.CompilerParams(needs_layout_passes=False),
    )
    return f(src_p, dst_p, comb0, comb1, hcat)


def _tc_b_body(u_ref, d_ref, b1_ref, w2_ref, as2_ref, ad2_ref,
               h2_ref, ta2_ref, tb2_ref):
    i = pl.program_id(0)
    u = u_ref[...]
    h1 = jnp.concatenate([u[0], u[1]], axis=-1)
    d = d_ref[...]
    den8 = jnp.concatenate([d[0][:, 4:8], d[1][:, 4:8]], axis=-1)
    drep = jnp.broadcast_to(den8[:, :, None], (BN_B, 8, 32)).reshape(BN_B, 256)
    h1 = h1 / (drep + 1e-16) + b1_ref[...]
    h1 = jnp.where(h1 > 0, h1, jnp.exp(h1) - 1.0)
    h2 = jnp.dot(h1, w2_ref[...], preferred_element_type=jnp.float32)
    h2_ref[...] = h2
    asrc2 = (h2[:, :16] * as2_ref[...]).sum(-1)
    adst2 = (h2[:, :16] * ad2_ref[...]).sum(-1)
    col = lax.broadcasted_iota(jnp.int32, (BN_B, 16), 1)
    rown = lax.broadcasted_iota(jnp.int32, (BN_B, 16), 0) + i * BN_B
    ta2_ref[...] = jnp.where(col == 0, asrc2[:, None], NEG)
    tb2_ref[...] = jnp.where(col == 0,
                             jnp.where(rown < 10000, adst2[:, None], NEG),
                             0.0)


def _tc_b(u1, den1, b1r, w2p, as2v, ad2v):
    grid = (N_PAD // BN_B,)
    return pl.pallas_call(
        _tc_b_body,
        grid=grid,
        in_specs=[
            pl.BlockSpec((2, BN_B, 128), lambda i: (0, i, 0)),
            pl.BlockSpec((2, BN_B, 8), lambda i: (0, i, 0)),
            pl.BlockSpec((1, 256), lambda i: (0, 0)),
            pl.BlockSpec((256, 128), lambda i: (0, 0)),
            pl.BlockSpec((1, 16), lambda i: (0, 0)),
            pl.BlockSpec((1, 16), lambda i: (0, 0)),
        ],
        out_specs=[
            pl.BlockSpec((BN_B, 128), lambda i: (i, 0)),
            pl.BlockSpec((BN_B, 16), lambda i: (i, 0)),
            pl.BlockSpec((BN_B, 16), lambda i: (i, 0)),
        ],
        out_shape=[
            jax.ShapeDtypeStruct((N_PAD, 128), jnp.float32),
            jax.ShapeDtypeStruct((N_PAD, 16), jnp.float32),
            jax.ShapeDtypeStruct((N_PAD, 16), jnp.float32),
        ],
    )(u1, den1, b1r, w2p, as2v, ad2v)


def _sc2_body(src_ref, dst_ref, as2_ref, ad2_ref, h2_ref,
              u2_out, d2_out,
              u2_sp, d2_sp, asrc_t, adst_t, sidx, didx, hrows, h16,
              exf2, z1d, zidx, sem):
    core = lax.axis_index("c")
    sub = lax.axis_index("s")
    half = src_ref.shape[0] // 2
    T = half // 16
    rpt = N_PAD // 16
    rs = sub * rpt
    iota16 = lax.broadcasted_iota(jnp.int32, (16,), 0)
    zero16 = jnp.zeros((16,), jnp.float32)
    zi16 = jnp.zeros((16,), jnp.int32)

    pltpu.sync_copy(as2_ref, asrc_t)
    pltpu.sync_copy(ad2_ref, adst_t)

    def zz(i, _):
        z1d[pl.ds(16 * i, 16)] = zero16
        return 0
    lax.fori_loop(0, ZR // 16, zz, 0)

    def zsrc(r, _):
        h16[r, :] = zero16
        return 0
    lax.fori_loop(0, ZR, zsrc, 0)

    def zscat(k, _):
        b = rs + k * ZR
        for v in range(ZR // 16):
            zidx[pl.ds(16 * v, 16)] = iota16 + (b + 16 * v)
        pltpu.sync_copy(h16.at[pl.ds(0, ZR)], u2_sp.at[zidx])
        pltpu.sync_copy(z1d, d2_sp.at[zidx])
        return 0
    lax.fori_loop(0, rpt // ZR, zscat, 0)
    plsc.subcore_barrier()

    def chunk(k, _):
        base = core * half + sub * T + k * C2
        pltpu.sync_copy(src_ref.at[pl.ds(base, C2)], sidx)
        pltpu.sync_copy(dst_ref.at[pl.ds(base, C2)], didx)
        cg = pltpu.async_copy(h2_ref.at[sidx], hrows, sem)

        def sixteen(i, _):
            sid = sidx[pl.ds(16 * i, 16)]
            did = didx[pl.ds(16 * i, 16)]
            a = plsc.load_gather(asrc_t, [sid])
            b = plsc.load_gather(adst_t, [did])
            s = a + b
            ex = jnp.exp(jnp.maximum(s, 0.2 * s))
            exf2[pl.ds(16 * i, 16)] = ex
            return 0
        lax.fori_loop(0, C2 // 16, sixteen, 0)
        cg.wait()

        def edge(r, _):
            w = plsc.load_gather(exf2, [zi16 + r])
            h16[r, :] = hrows[r, pl.ds(0, 16)] * w
            return 0
        lax.fori_loop(0, C2, edge, 0)

        pltpu.sync_copy(exf2, d2_sp.at[didx], add=True)
        pltpu.sync_copy(h16, u2_sp.at[didx], add=True)
        return 0

    lax.fori_loop(0, T // C2, chunk, 0)
    plsc.subcore_barrier()

    pltpu.sync_copy(u2_sp.at[pl.ds(rs, rpt)],
                    u2_out.at[core, pl.ds(rs, rpt)])
    pltpu.sync_copy(d2_sp.at[pl.ds(rs, rpt)],
                    d2_out.at[core, pl.ds(rs, rpt)])


def _sc2(src_p, dst_p, ta2f, tb2f, h2p):
    mesh = plsc.VectorSubcoreMesh(core_axis_name="c", subcore_axis_name="s")
    f = pl.kernel(
        _sc2_body,
        out_type=[
            jax.ShapeDtypeStruct((2, N_PAD, 16), jnp.float32),
            jax.ShapeDtypeStruct((2, N_PAD), jnp.float32),
        ],
        mesh=mesh,
        scratch_types=[
            pltpu.VMEM_SHARED((N_PAD, 16), jnp.float32),
            pltpu.VMEM_SHARED((N_PAD,), jnp.float32),
            pltpu.VMEM((N_PAD,), jnp.float32),
            pltpu.VMEM((N_PAD,), jnp.float32),
            pltpu.VMEM((C2,), jnp.int32),
            pltpu.VMEM((C2,), jnp.int32),
            pltpu.VMEM((C2, 128), jnp.float32),
            pltpu.VMEM((C2, 16), jnp.float32),
            pltpu.VMEM((C2,), jnp.float32),
            pltpu.VMEM((ZR,), jnp.float32),
            pltpu.VMEM((ZR,), jnp.int32),
            pltpu.SemaphoreType.DMA,
        ],
        compiler_params=pltpu.CompilerParams(needs_layout_passes=False),
    )
    return f(src_p, dst_p, ta2f, tb2f, h2p)


def _tc_c_body(u_ref, d_ref, b2_ref, o_ref):
    u = u_ref[...]
    d = d_ref[...]
    usum = u[0] + u[1]
    dsum = d[0][:, 0:1] + d[1][:, 0:1]
    o_ref[...] = usum / (dsum + 1e-16) + b2_ref[...]


def _tc_c(u2p, d2p, b2r):
    grid = (N_PAD // BN_C,)
    return pl.pallas_call(
        _tc_c_body,
        grid=grid,
        in_specs=[
            pl.BlockSpec((2, BN_C, 16), lambda i: (0, i, 0)),
            pl.BlockSpec((2, BN_C, 1), lambda i: (0, i, 0)),
            pl.BlockSpec((1, 16), lambda i: (0, 0)),
        ],
        out_specs=pl.BlockSpec((BN_C, 16), lambda i: (i, 0)),
        out_shape=jax.ShapeDtypeStruct((N_PAD, 16), jnp.float32),
    )(u2p, d2p, b2r)


def kernel(x, edge_index, W1, att_src1, att_dst1, b1, W2, att_src2,
           att_dst2, b2):
    N = x.shape[0]
    E0 = edge_index.shape[1]
    loops = jnp.arange(N, dtype=edge_index.dtype)
    ei = jnp.concatenate([edge_index, jnp.stack([loops, loops])], axis=1)
    src, dst = ei[0], ei[1]
    E = E0 + N
    step = 16 * C1
    e_pad = ((E + step - 1) // step) * step
    assert e_pad % (32 * C2) == 0

    src_p = jnp.concatenate([src, jnp.zeros((e_pad - E,), jnp.int32)])
    dst_p = jnp.concatenate([dst, jnp.full((e_pad - E,), N, jnp.int32)])

    x_p = jnp.zeros((N_PAD, 768), jnp.float32).at[:N, :767].set(x)
    w1p = jnp.zeros((768, 256), jnp.float32).at[:767].set(W1)
    h_split, a_src2d, a_dst2d = _tc_a(x_p, w1p, att_src1.reshape(2, 4, 32),
                                      att_dst1.reshape(2, 4, 32))
    hcat = h_split.reshape(2 * N_PAD, 128)
    ad_neg = a_dst2d.at[:, N:, :].set(NEG)
    comb0 = jnp.concatenate([a_src2d[0], ad_neg[0]], axis=-1).reshape(-1)
    comb1 = jnp.concatenate([a_src2d[1], ad_neg[1]], axis=-1).reshape(-1)

    u1, den1 = _sc1(src_p, dst_p, comb0, comb1, hcat)
    den1r = den1.reshape(2, N_PAD, 8)

    w2p = jnp.zeros((256, 128), jnp.float32).at[:, :10].set(W2)
    as2v = jnp.zeros((1, 16), jnp.float32).at[0, :10].set(att_src2.reshape(10))
    ad2v = jnp.zeros((1, 16), jnp.float32).at[0, :10].set(att_dst2.reshape(10))
    h2p, ta2, tb2 = _tc_b(u1, den1r, b1.reshape(1, 256), w2p, as2v, ad2v)

    u2p, d2p = _sc2(src_p, dst_p, ta2[:, 0], tb2[:, 0], h2p)

    b2r = jnp.zeros((1, 16), jnp.float32).at[0, :10].set(b2)
    outp = _tc_c(u2p, d2p[:, :, None], b2r)
    return outp[:N, :10]
